# Initial kernel scaffold; baseline (speedup 1.0000x reference)
import jax
import jax.numpy as jnp
from jax.experimental import pallas as pl


def kernel(X, edge_index, edge_weight, batch, mlp_W1, mlp_b1, mlp_W2, mlp_b2, mlp_W3, mlp_b3, F1, F2, att_W1, att_b1, att_W2, fc_W, fc_b, gfc_W, gfc_b, out_W, out_b):
    raise NotImplementedError("write your pallas kernel here")


# speedup vs baseline: 4.9246x; 4.9246x over previous
import functools

import jax
import jax.numpy as jnp
from jax import lax
from jax.experimental import pallas as pl
from jax.experimental.pallas import tpu as pltpu
from jax.experimental.pallas import tpu_sc as plsc

N_NODES = 10000
N_EDGES = 320000
NHID = 32
NUM_GRAPHS = 128
MAX_ITER = 10
GAMMA = 0.8
EPS_F = 1e-6
EDGE_CHUNK = 16000


def _spmm_sc(zfm, src, dst, w):
  mesh = plsc.VectorSubcoreMesh(core_axis_name="c", subcore_axis_name="s")

  @functools.partial(
      pl.kernel,
      mesh=mesh,
      compiler_params=pltpu.CompilerParams(needs_layout_passes=False),
      out_type=jax.ShapeDtypeStruct((NHID, N_NODES), jnp.float32),
      scratch_types=[
          pltpu.VMEM((N_NODES,), jnp.float32),
          pltpu.VMEM((N_NODES,), jnp.float32),
          pltpu.VMEM((EDGE_CHUNK,), jnp.int32),
          pltpu.VMEM((EDGE_CHUNK,), jnp.int32),
          pltpu.VMEM((EDGE_CHUNK,), jnp.float32),
      ],
  )
  def k(z_hbm, src_hbm, dst_hbm, w_hbm, out_hbm, zin_v, zout_v, src_v, dst_v,
        w_v):
    wid = lax.axis_index("s") * 2 + lax.axis_index("c")
    pltpu.sync_copy(z_hbm.at[wid], zin_v)

    def zbody(i, _):
      zout_v[pl.ds(i * 16, 16)] = jnp.zeros((16,), jnp.float32)
      return 0

    lax.fori_loop(0, N_NODES // 16, zbody, 0)

    def cbody(c, _):
      base = c * EDGE_CHUNK
      pltpu.sync_copy(src_hbm.at[pl.ds(base, EDGE_CHUNK)], src_v)
      pltpu.sync_copy(dst_hbm.at[pl.ds(base, EDGE_CHUNK)], dst_v)
      pltpu.sync_copy(w_hbm.at[pl.ds(base, EDGE_CHUNK)], w_v)

      def ebody(i, _):
        off = i * 64
        for u in range(4):
          s = src_v[pl.ds(off + u * 16, 16)]
          d = dst_v[pl.ds(off + u * 16, 16)]
          ww = w_v[pl.ds(off + u * 16, 16)]
          vals = plsc.load_gather(zin_v, [s]) * ww
          plsc.addupdate_scatter(zout_v, [d], vals)
        return 0

      lax.fori_loop(0, EDGE_CHUNK // 64, ebody, 0)
      return 0

    lax.fori_loop(0, N_EDGES // EDGE_CHUNK, cbody, 0)
    pltpu.sync_copy(zout_v, out_hbm.at[wid])

  return k(zfm, src, dst, w)


def _mlp_kernel(x_ref, w1_ref, b1_ref, w2_ref, b2_ref, w3_ref, b3_ref,
                out_ref):
  h1 = jax.nn.relu(
      jnp.dot(w1_ref[...], x_ref[...], preferred_element_type=jnp.float32)
      + b1_ref[...])
  h2 = jax.nn.relu(
      jnp.dot(w2_ref[...], h1, preferred_element_type=jnp.float32)
      + b2_ref[...])
  out_ref[...] = (
      jnp.dot(w3_ref[...], h2, preferred_element_type=jnp.float32)
      + b3_ref[...])


def _step_kernel(f_ref, r_ref, xh_ref, out_ref):
  ff = jnp.dot(f_ref[...].T, f_ref[...], preferred_element_type=jnp.float32)
  norm = jnp.sqrt(jnp.sum(ff * ff))
  gf = (GAMMA / (norm + EPS_F)) * ff
  z = r_ref[...] + xh_ref[...]
  out_ref[...] = jnp.dot(gf, z, preferred_element_type=jnp.float32)


def _head_kernel(r1_ref, r2_ref, xh_ref, batch_ref, aw1_ref, ab1_ref,
                 aw2_ref, fcw_ref, fcb_ref, gfcw_ref, gfcb_ref, outw_ref,
                 outb_ref, out_ref):
  xh = xh_ref[...]
  z1 = r1_ref[...] + xh
  z2 = r2_ref[...] + xh
  aw1 = aw1_ref[...]
  ab1 = ab1_ref[...]
  t1 = jnp.tanh(jnp.dot(aw1, z1, preferred_element_type=jnp.float32) + ab1)
  t2 = jnp.tanh(jnp.dot(aw1, z2, preferred_element_type=jnp.float32) + ab1)
  a1 = jnp.dot(aw2_ref[...], t1, preferred_element_type=jnp.float32)
  a2 = jnp.dot(aw2_ref[...], t2, preferred_element_type=jnp.float32)
  m = jnp.maximum(a1, a2)
  e1 = jnp.exp(a1 - m)
  e2 = jnp.exp(a2 - m)
  inv = 1.0 / (e1 + e2)
  o = z1 * (e1 * inv) + z2 * (e2 * inv)
  o = jax.nn.relu(
      jnp.dot(fcw_ref[...], o, preferred_element_type=jnp.float32)
      + fcb_ref[...])
  gid = lax.broadcasted_iota(jnp.int32, (NUM_GRAPHS, N_NODES), 0)
  onehot = (gid == batch_ref[...]).astype(jnp.float32)
  pooled = lax.dot_general(onehot, o, (((1,), (1,)), ((), ())),
                           preferred_element_type=jnp.float32)
  pooled = jax.nn.relu(
      jnp.dot(pooled, gfcw_ref[...].T, preferred_element_type=jnp.float32)
      + gfcb_ref[...])
  logits = (jnp.dot(pooled, outw_ref[...].T,
                    preferred_element_type=jnp.float32) + outb_ref[...])
  mx = jnp.max(logits, axis=1, keepdims=True)
  lse = mx + jnp.log(jnp.sum(jnp.exp(logits - mx), axis=1, keepdims=True))
  out_ref[...] = logits - lse


def kernel(X, edge_index, edge_weight, batch, mlp_W1, mlp_b1, mlp_W2, mlp_b2,
           mlp_W3, mlp_b3, F1, F2, att_W1, att_b1, att_W2, fc_W, fc_b, gfc_W,
           gfc_b, out_W, out_b):
  src = edge_index[0].astype(jnp.int32)
  dst = edge_index[1].astype(jnp.int32)
  w = edge_weight.astype(jnp.float32)
  batch2d = batch.astype(jnp.int32).reshape(1, N_NODES)

  fm = jax.ShapeDtypeStruct((NHID, N_NODES), jnp.float32)

  mlp = pl.pallas_call(
      _mlp_kernel, out_shape=jax.ShapeDtypeStruct((NHID, N_NODES),
                                                  jnp.float32))
  Xh = mlp(X, mlp_W1, mlp_b1.reshape(-1, 1), mlp_W2, mlp_b2.reshape(-1, 1),
           mlp_W3, mlp_b3.reshape(-1, 1))

  step = pl.pallas_call(_step_kernel, out_shape=fm)

  def branch(F, k):
    R = jnp.zeros((NHID, N_NODES), jnp.float32)
    for _ in range(MAX_ITER - 1):
      P = step(F, R, Xh)
      for _ in range(k):
        P = _spmm_sc(P, src, dst, w)
      R = P
    return R

  R1 = branch(F1, 1)
  R2 = branch(F2, 2)

  head = pl.pallas_call(
      _head_kernel,
      out_shape=jax.ShapeDtypeStruct((NUM_GRAPHS, out_W.shape[0]),
                                     jnp.float32))
  return head(R1, R2, Xh, batch2d, att_W1, att_b1.reshape(-1, 1), att_W2,
              fc_W, fc_b.reshape(-1, 1), gfc_W, gfc_b.reshape(1, -1), out_W,
              out_b.reshape(1, -1))


# speedup vs baseline: 14.9675x; 3.0393x over previous
import functools

import jax
import jax.numpy as jnp
from jax import lax
from jax.experimental import pallas as pl
from jax.experimental.pallas import tpu as pltpu
from jax.experimental.pallas import tpu_sc as plsc

N_NODES = 10000
N_EDGES = 320000
NHID = 32
NUM_GRAPHS = 128
MAX_ITER = 10
GAMMA = 0.8
EPS_F = 1e-6
EDGE_CHUNK = 16000


def _spmm_sc(zfm, sd_packed, w):
  mesh = plsc.VectorSubcoreMesh(core_axis_name="c", subcore_axis_name="s")
  nch = N_EDGES // EDGE_CHUNK

  @functools.partial(
      pl.kernel,
      mesh=mesh,
      compiler_params=pltpu.CompilerParams(needs_layout_passes=False),
      out_type=jax.ShapeDtypeStruct((NHID, N_NODES), jnp.float32),
      scratch_types=[
          pltpu.VMEM((N_NODES,), jnp.float32),
          pltpu.VMEM((N_NODES,), jnp.float32),
          pltpu.VMEM((2, EDGE_CHUNK), jnp.int32),
          pltpu.VMEM((2, EDGE_CHUNK), jnp.float32),
          pltpu.SemaphoreType.DMA,
          pltpu.SemaphoreType.DMA,
          pltpu.SemaphoreType.DMA,
          pltpu.SemaphoreType.DMA,
          pltpu.SemaphoreType.DMA,
      ],
  )
  def k(z_hbm, sd_hbm, w_hbm, out_hbm, zin_v, zout_v, sd_v, w_v, sem_sd0,
        sem_sd1, sem_w0, sem_w1, sem_z):
    wid = lax.axis_index("s") * 2 + lax.axis_index("c")
    sem_sd = (sem_sd0, sem_sd1)
    sem_w = (sem_w0, sem_w1)

    def start(c, b):
      pltpu.async_copy(sd_hbm.at[pl.ds(c * EDGE_CHUNK, EDGE_CHUNK)],
                       sd_v.at[b], sem_sd[b])
      pltpu.async_copy(w_hbm.at[pl.ds(c * EDGE_CHUNK, EDGE_CHUNK)],
                       w_v.at[b], sem_w[b])

    def wait(b):
      pltpu.make_async_copy(sd_hbm.at[pl.ds(0, EDGE_CHUNK)], sd_v.at[b],
                            sem_sd[b]).wait()
      pltpu.make_async_copy(w_hbm.at[pl.ds(0, EDGE_CHUNK)], w_v.at[b],
                            sem_w[b]).wait()

    def compute(b):
      @plsc.parallel_loop(0, EDGE_CHUNK // 16, unroll=8)
      def _(i):
        u = sd_v[b, pl.ds(i * 16, 16)]
        s = jnp.bitwise_and(u, 0xFFFF)
        d = lax.shift_right_logical(u, 16)
        vals = plsc.load_gather(zin_v, [s]) * w_v[b, pl.ds(i * 16, 16)]
        plsc.addupdate_scatter(zout_v, [d], vals)

    zcp = pltpu.async_copy(z_hbm.at[wid], zin_v, sem_z)
    start(0, 0)

    @plsc.parallel_loop(0, N_NODES // 16, unroll=8)
    def _(i):
      zout_v[pl.ds(i * 16, 16)] = jnp.zeros((16,), jnp.float32)

    zcp.wait()

    def cbody(i, _):
      c0 = 2 * i
      start(c0 + 1, 1)
      wait(0)
      compute(0)
      start(c0 + 2, 0)
      wait(1)
      compute(1)
      return 0

    lax.fori_loop(0, nch // 2 - 1, cbody, 0)
    start(nch - 1, 1)
    wait(0)
    compute(0)
    wait(1)
    compute(1)
    pltpu.sync_copy(zout_v, out_hbm.at[wid])

  return k(zfm, sd_packed, w)


def _mlp_kernel(x_ref, w1_ref, b1_ref, w2_ref, b2_ref, w3_ref, b3_ref,
                out_ref):
  h1 = jax.nn.relu(
      jnp.dot(w1_ref[...], x_ref[...], preferred_element_type=jnp.float32)
      + b1_ref[...])
  h2 = jax.nn.relu(
      jnp.dot(w2_ref[...], h1, preferred_element_type=jnp.float32)
      + b2_ref[...])
  out_ref[...] = (
      jnp.dot(w3_ref[...], h2, preferred_element_type=jnp.float32)
      + b3_ref[...])


def _step_kernel(f_ref, r_ref, xh_ref, out_ref):
  ff = jnp.dot(f_ref[...].T, f_ref[...], preferred_element_type=jnp.float32)
  norm = jnp.sqrt(jnp.sum(ff * ff))
  gf = (GAMMA / (norm + EPS_F)) * ff
  z = r_ref[...] + xh_ref[...]
  out_ref[...] = jnp.dot(gf, z, preferred_element_type=jnp.float32)


def _head_kernel(r1_ref, r2_ref, xh_ref, batch_ref, aw1_ref, ab1_ref,
                 aw2_ref, fcw_ref, fcb_ref, gfcw_ref, gfcb_ref, outw_ref,
                 outb_ref, out_ref):
  xh = xh_ref[...]
  z1 = r1_ref[...] + xh
  z2 = r2_ref[...] + xh
  aw1 = aw1_ref[...]
  ab1 = ab1_ref[...]
  t1 = jnp.tanh(jnp.dot(aw1, z1, preferred_element_type=jnp.float32) + ab1)
  t2 = jnp.tanh(jnp.dot(aw1, z2, preferred_element_type=jnp.float32) + ab1)
  a1 = jnp.dot(aw2_ref[...], t1, preferred_element_type=jnp.float32)
  a2 = jnp.dot(aw2_ref[...], t2, preferred_element_type=jnp.float32)
  m = jnp.maximum(a1, a2)
  e1 = jnp.exp(a1 - m)
  e2 = jnp.exp(a2 - m)
  inv = 1.0 / (e1 + e2)
  o = z1 * (e1 * inv) + z2 * (e2 * inv)
  o = jax.nn.relu(
      jnp.dot(fcw_ref[...], o, preferred_element_type=jnp.float32)
      + fcb_ref[...])
  gid = lax.broadcasted_iota(jnp.int32, (NUM_GRAPHS, N_NODES), 0)
  onehot = (gid == batch_ref[...]).astype(jnp.float32)
  pooled = lax.dot_general(onehot, o, (((1,), (1,)), ((), ())),
                           preferred_element_type=jnp.float32)
  pooled = jax.nn.relu(
      jnp.dot(pooled, gfcw_ref[...].T, preferred_element_type=jnp.float32)
      + gfcb_ref[...])
  logits = (jnp.dot(pooled, outw_ref[...].T,
                    preferred_element_type=jnp.float32) + outb_ref[...])
  mx = jnp.max(logits, axis=1, keepdims=True)
  lse = mx + jnp.log(jnp.sum(jnp.exp(logits - mx), axis=1, keepdims=True))
  out_ref[...] = logits - lse


def kernel(X, edge_index, edge_weight, batch, mlp_W1, mlp_b1, mlp_W2, mlp_b2,
           mlp_W3, mlp_b3, F1, F2, att_W1, att_b1, att_W2, fc_W, fc_b, gfc_W,
           gfc_b, out_W, out_b):
  src = edge_index[0].astype(jnp.int32)
  dst = edge_index[1].astype(jnp.int32)
  sd = jnp.bitwise_or(src, jnp.left_shift(dst, 16))
  w = edge_weight.astype(jnp.float32)
  batch2d = batch.astype(jnp.int32).reshape(1, N_NODES)

  fm = jax.ShapeDtypeStruct((NHID, N_NODES), jnp.float32)

  mlp = pl.pallas_call(
      _mlp_kernel, out_shape=jax.ShapeDtypeStruct((NHID, N_NODES),
                                                  jnp.float32))
  Xh = mlp(X, mlp_W1, mlp_b1.reshape(-1, 1), mlp_W2, mlp_b2.reshape(-1, 1),
           mlp_W3, mlp_b3.reshape(-1, 1))

  step = pl.pallas_call(_step_kernel, out_shape=fm)

  def branch(F, k):
    R = jnp.zeros((NHID, N_NODES), jnp.float32)
    for _ in range(MAX_ITER - 1):
      P = step(F, R, Xh)
      for _ in range(k):
        P = _spmm_sc(P, sd, w)
      R = P
    return R

  R1 = branch(F1, 1)
  R2 = branch(F2, 2)

  head = pl.pallas_call(
      _head_kernel,
      out_shape=jax.ShapeDtypeStruct((NUM_GRAPHS, out_W.shape[0]),
                                     jnp.float32))
  return head(R1, R2, Xh, batch2d, att_W1, att_b1.reshape(-1, 1), att_W2,
              fc_W, fc_b.reshape(-1, 1), gfc_W, gfc_b.reshape(1, -1), out_W,
              out_b.reshape(1, -1))
